# baseline (speedup 1.0000x reference)
import functools

import jax
import jax.numpy as jnp
import numpy as np
from jax import lax
from jax.experimental import pallas as pl
from jax.experimental.pallas import tpu as pltpu
from jax.experimental.pallas import tpu_sc as plsc

_BATCH = 16384
_TOTAL = 2600
_SEG = 100
_NHEAD = 26
_HPAD = 32

_B_SC = 4096
_B_TC = _BATCH - _B_SC

_NWORKERS = 32
_SC_ROWS = _B_SC * _NHEAD
_ROWS_PER_W = _SC_ROWS // _NWORKERS
_CHUNK_ROWS = 256
_CHUNK_WORDS = _CHUNK_ROWS * _SEG
_NCHUNKS = _ROWS_PER_W // _CHUNK_ROWS
_GROUPS = _CHUNK_ROWS // 16

_LN2 = 0.6931471805599453
_SQRT2 = 1.4142135623730951

_BLK = 1024
_IND = np.zeros((_TOTAL, _HPAD), np.float32)
for _j in range(_TOTAL):
    _IND[_j, _j // _SEG] = 1.0
_INDT = _IND.T.copy()


def _vlog(s):
    bits = plsc.bitcast(s, jnp.int32)
    e = lax.shift_right_arithmetic(bits, 23) - 127
    mbits = jnp.bitwise_or(jnp.bitwise_and(bits, 0x007FFFFF), 0x3F800000)
    m = plsc.bitcast(mbits, jnp.float32)
    big = m > _SQRT2
    m = jnp.where(big, m * 0.5, m)
    e = (e + jnp.where(big, 1, 0)).astype(jnp.float32)
    t = (m - 1.0) / (m + 1.0)
    w = t * t
    p = 2.0 * t * (1.0 + w * (1.0 / 3.0 + w * (0.2 + w * (1.0 / 7.0 + w * (1.0 / 9.0)))))
    return e * _LN2 + p


def _compute_chunk(buf, iota16):
    def group_body(g, carry):
        base = g * (16 * _SEG)
        idx0 = base + iota16 * _SEG

        acc = [jnp.full((16,), -jnp.inf, jnp.float32) for _ in range(4)]
        for j in range(_SEG):
            v = plsc.load_gather(buf, [idx0 + j])
            acc[j % 4] = jnp.maximum(acc[j % 4], v)
        mx = jnp.maximum(jnp.maximum(acc[0], acc[1]),
                         jnp.maximum(acc[2], acc[3]))

        sacc = [jnp.zeros((16,), jnp.float32) for _ in range(4)]
        for j in range(_SEG):
            v = plsc.load_gather(buf, [idx0 + j])
            sacc[j % 4] = sacc[j % 4] + jnp.exp(v - mx)
        s = (sacc[0] + sacc[1]) + (sacc[2] + sacc[3])

        c = mx + _vlog(s)

        for j in range(_SEG):
            idx = idx0 + j
            v = plsc.load_gather(buf, [idx])
            plsc.store_scatter(buf, [idx], v - c)
        return carry

    lax.fori_loop(0, _GROUPS, group_body, 0)


def _sc_body(x_hbm, out_hbm, buf, sem_in, sem_out):
    wid = lax.axis_index("s") * 2 + lax.axis_index("c")
    wbase = wid * (_ROWS_PER_W * _SEG)
    iota16 = lax.iota(jnp.int32, 16)

    def chunk_body(t, carry):
        base = wbase + t * _CHUNK_WORDS
        pltpu.async_copy(x_hbm.at[pl.ds(base, _CHUNK_WORDS)], buf, sem_in).wait()
        _compute_chunk(buf, iota16)
        pltpu.async_copy(buf, out_hbm.at[pl.ds(base, _CHUNK_WORDS)], sem_out).wait()
        return carry

    lax.fori_loop(0, _NCHUNKS, chunk_body, 0)


def _sc_call(x):
    call = functools.partial(
        pl.kernel,
        out_type=jax.ShapeDtypeStruct((_BATCH * _TOTAL,), jnp.float32),
        mesh=plsc.VectorSubcoreMesh(core_axis_name="c", subcore_axis_name="s"),
        scratch_types=[
            pltpu.VMEM((_CHUNK_WORDS,), jnp.float32),
            pltpu.SemaphoreType.DMA,
            pltpu.SemaphoreType.DMA,
        ],
        compiler_params=pltpu.CompilerParams(needs_layout_passes=False),
    )(_sc_body)
    return call(x.reshape(_BATCH * _TOTAL))


def _tc_body(x_ref, o1_ref, ind_ref, indt_ref, o_ref):
    del o1_ref
    x = x_ref[...]
    m = jnp.max(x, axis=1, keepdims=True)
    e = jnp.exp(x - m)
    s = jax.lax.dot_general(e, ind_ref[...], (((1,), (0,)), ((), ())),
                            preferred_element_type=jnp.float32)
    l = jnp.log(jnp.maximum(s, 1e-30))
    lb = jax.lax.dot_general(l, indt_ref[...], (((1,), (0,)), ((), ())),
                             preferred_element_type=jnp.float32)
    o_ref[...] = x - m - lb


def _tc_call(x, o1):
    off = _B_SC // _BLK
    return pl.pallas_call(
        _tc_body,
        out_shape=jax.ShapeDtypeStruct((_BATCH, _TOTAL), jnp.float32),
        grid=(_B_TC // _BLK,),
        in_specs=[
            pl.BlockSpec((_BLK, _TOTAL), lambda i: (i + off, 0)),
            pl.BlockSpec(memory_space=pl.ANY),
            pl.BlockSpec((_TOTAL, _HPAD), lambda i: (0, 0)),
            pl.BlockSpec((_HPAD, _TOTAL), lambda i: (0, 0)),
        ],
        out_specs=pl.BlockSpec((_BLK, _TOTAL), lambda i: (i + off, 0)),
        input_output_aliases={1: 0},
        compiler_params=pltpu.CompilerParams(
            dimension_semantics=("arbitrary",)),
    )(x, o1, jnp.asarray(_IND), jnp.asarray(_INDT))


@jax.jit
def kernel(logits):
    o1 = _sc_call(logits).reshape(_BATCH, _TOTAL)
    return _tc_call(logits, o1)


# speedup vs baseline: 1.1321x; 1.1321x over previous
import functools

import jax
import jax.numpy as jnp
import numpy as np
from jax import lax
from jax.experimental import pallas as pl
from jax.experimental.pallas import tpu as pltpu
from jax.experimental.pallas import tpu_sc as plsc

_BATCH = 16384
_TOTAL = 2600
_SEG = 100
_NHEAD = 26
_HPAD = 32

_B_SC = 4096
_B_TC = _BATCH - _B_SC

_NWORKERS = 32
_SC_ROWS = _B_SC * _NHEAD
_ROWS_PER_W = _SC_ROWS // _NWORKERS
_CHUNK_ROWS = 256
_CHUNK_WORDS = _CHUNK_ROWS * _SEG
_NCHUNKS = _ROWS_PER_W // _CHUNK_ROWS
_GROUPS = _CHUNK_ROWS // 16

_LN2 = 0.6931471805599453
_SQRT2 = 1.4142135623730951

_BLK = 512
_IND = np.zeros((_TOTAL, _HPAD), np.float32)
for _j in range(_TOTAL):
    _IND[_j, _j // _SEG] = 1.0
_INDT = _IND.T.copy()


def _vlog(s):
    bits = plsc.bitcast(s, jnp.int32)
    e = lax.shift_right_arithmetic(bits, 23) - 127
    mbits = jnp.bitwise_or(jnp.bitwise_and(bits, 0x007FFFFF), 0x3F800000)
    m = plsc.bitcast(mbits, jnp.float32)
    big = m > _SQRT2
    m = jnp.where(big, m * 0.5, m)
    e = (e + jnp.where(big, 1, 0)).astype(jnp.float32)
    t = (m - 1.0) / (m + 1.0)
    w = t * t
    p = 2.0 * t * (1.0 + w * (1.0 / 3.0 + w * (0.2 + w * (1.0 / 7.0 + w * (1.0 / 9.0)))))
    return e * _LN2 + p


def _compute_chunk(buf, iota16):
    def group_body(g, carry):
        base = g * (16 * _SEG)
        idx0 = base + iota16 * _SEG

        acc = [jnp.full((16,), -jnp.inf, jnp.float32) for _ in range(4)]
        for j in range(_SEG):
            v = plsc.load_gather(buf, [idx0 + j])
            acc[j % 4] = jnp.maximum(acc[j % 4], v)
        mx = jnp.maximum(jnp.maximum(acc[0], acc[1]),
                         jnp.maximum(acc[2], acc[3]))

        sacc = [jnp.zeros((16,), jnp.float32) for _ in range(4)]
        for j in range(_SEG):
            v = plsc.load_gather(buf, [idx0 + j])
            sacc[j % 4] = sacc[j % 4] + jnp.exp(v - mx)
        s = (sacc[0] + sacc[1]) + (sacc[2] + sacc[3])

        c = mx + _vlog(s)

        for j in range(_SEG):
            idx = idx0 + j
            v = plsc.load_gather(buf, [idx])
            plsc.store_scatter(buf, [idx], v - c)
        return carry

    lax.fori_loop(0, _GROUPS, group_body, 0)


def _sc_body(x_hbm, out_hbm, buf, sem_in, sem_out):
    wid = lax.axis_index("s") * 2 + lax.axis_index("c")
    wbase = wid * (_ROWS_PER_W * _SEG)
    iota16 = lax.iota(jnp.int32, 16)

    def chunk_body(t, carry):
        base = wbase + t * _CHUNK_WORDS
        pltpu.async_copy(x_hbm.at[pl.ds(base, _CHUNK_WORDS)], buf, sem_in).wait()
        _compute_chunk(buf, iota16)
        pltpu.async_copy(buf, out_hbm.at[pl.ds(base, _CHUNK_WORDS)], sem_out).wait()
        return carry

    lax.fori_loop(0, _NCHUNKS, chunk_body, 0)


def _sc_call(x_sc_flat):
    call = functools.partial(
        pl.kernel,
        out_type=jax.ShapeDtypeStruct((_B_SC * _TOTAL,), jnp.float32),
        mesh=plsc.VectorSubcoreMesh(core_axis_name="c", subcore_axis_name="s"),
        scratch_types=[
            pltpu.VMEM((_CHUNK_WORDS,), jnp.float32),
            pltpu.SemaphoreType.DMA,
            pltpu.SemaphoreType.DMA,
        ],
        compiler_params=pltpu.CompilerParams(needs_layout_passes=False),
    )(_sc_body)
    return call(x_sc_flat)


def _tc_body(x_ref, ind_ref, indt_ref, o_ref):
    x = x_ref[...]
    m = jnp.max(x, axis=1, keepdims=True)
    e = jnp.exp(x - m)
    s = jax.lax.dot_general(e, ind_ref[...], (((1,), (0,)), ((), ())),
                            preferred_element_type=jnp.float32,
                            precision=jax.lax.Precision.HIGHEST)
    l = jnp.log(jnp.maximum(s, 1e-30))
    lb = jax.lax.dot_general(l, indt_ref[...], (((1,), (0,)), ((), ())),
                             preferred_element_type=jnp.float32,
                             precision=jax.lax.Precision.HIGHEST)
    o_ref[...] = x - m - lb


def _tc_call(x):
    off = _B_SC // _BLK
    return pl.pallas_call(
        _tc_body,
        out_shape=jax.ShapeDtypeStruct((_BATCH, _TOTAL), jnp.float32),
        grid=(_B_TC // _BLK,),
        in_specs=[
            pl.BlockSpec((_BLK, _TOTAL), lambda i: (i + off, 0)),
            pl.BlockSpec((_TOTAL, _HPAD), lambda i: (0, 0)),
            pl.BlockSpec((_HPAD, _TOTAL), lambda i: (0, 0)),
        ],
        out_specs=pl.BlockSpec((_BLK, _TOTAL), lambda i: (i + off, 0)),
        compiler_params=pltpu.CompilerParams(
            dimension_semantics=("arbitrary",)),
    )(x, jnp.asarray(_IND), jnp.asarray(_INDT))


@jax.jit
def kernel(logits):
    sc_out = _sc_call(logits[:_B_SC].reshape(_B_SC * _TOTAL))
    tc_out = _tc_call(logits)
    return lax.dynamic_update_slice(
        tc_out, sc_out.reshape(_B_SC, _TOTAL), (0, 0))


# speedup vs baseline: 1.9868x; 1.7549x over previous
import functools

import jax
import jax.numpy as jnp
import numpy as np
from jax import lax
from jax.experimental import pallas as pl
from jax.experimental.pallas import tpu as pltpu
from jax.experimental.pallas import tpu_sc as plsc

_BATCH = 16384
_TOTAL = 2600
_SEG = 100
_NHEAD = 26
_HPAD = 32

_B_SC = 2048
_B_TC = _BATCH - _B_SC

_NWORKERS = 32
_SC_ROWS = _B_SC * _NHEAD
_ROWS_PER_W = _SC_ROWS // _NWORKERS
_CHUNK_ROWS = 256
_CHUNK_WORDS = _CHUNK_ROWS * _SEG
_NCHUNKS = _ROWS_PER_W // _CHUNK_ROWS
_GROUPS = _CHUNK_ROWS // 16

_LN2 = 0.6931471805599453
_SQRT2 = 1.4142135623730951

_BLK = 1024
_IND = np.zeros((_TOTAL, _HPAD), np.float32)
for _j in range(_TOTAL):
    _IND[_j, _j // _SEG] = 1.0
_INDT = _IND.T.copy()


def _vlog(s):
    bits = plsc.bitcast(s, jnp.int32)
    e = lax.shift_right_arithmetic(bits, 23) - 127
    mbits = jnp.bitwise_or(jnp.bitwise_and(bits, 0x007FFFFF), 0x3F800000)
    m = plsc.bitcast(mbits, jnp.float32)
    big = m > _SQRT2
    m = jnp.where(big, m * 0.5, m)
    e = (e + jnp.where(big, 1, 0)).astype(jnp.float32)
    t = (m - 1.0) / (m + 1.0)
    w = t * t
    p = 2.0 * t * (1.0 + w * (1.0 / 3.0 + w * (0.2 + w * (1.0 / 7.0 + w * (1.0 / 9.0)))))
    return e * _LN2 + p


def _compute_chunk(buf, iota16):
    def group_body(g, carry):
        base = g * (16 * _SEG)
        idx0 = base + iota16 * _SEG

        acc = [jnp.full((16,), -jnp.inf, jnp.float32) for _ in range(4)]
        for j in range(_SEG):
            v = plsc.load_gather(buf, [idx0 + j])
            acc[j % 4] = jnp.maximum(acc[j % 4], v)
        mx = jnp.maximum(jnp.maximum(acc[0], acc[1]),
                         jnp.maximum(acc[2], acc[3]))

        sacc = [jnp.zeros((16,), jnp.float32) for _ in range(4)]
        for j in range(_SEG):
            v = plsc.load_gather(buf, [idx0 + j])
            sacc[j % 4] = sacc[j % 4] + jnp.exp(v - mx)
        s = (sacc[0] + sacc[1]) + (sacc[2] + sacc[3])

        c = mx + _vlog(s)

        for j in range(_SEG):
            idx = idx0 + j
            v = plsc.load_gather(buf, [idx])
            plsc.store_scatter(buf, [idx], v - c)
        return carry

    lax.fori_loop(0, _GROUPS, group_body, 0)


def _sc_body(x_hbm, out_hbm, buf, sem_in, sem_out):
    wid = lax.axis_index("s") * 2 + lax.axis_index("c")
    wbase = wid * (_ROWS_PER_W * _SEG)
    iota16 = lax.iota(jnp.int32, 16)

    def chunk_body(t, carry):
        base = wbase + t * _CHUNK_WORDS
        pltpu.async_copy(x_hbm.at[pl.ds(base, _CHUNK_WORDS)], buf, sem_in).wait()
        _compute_chunk(buf, iota16)
        pltpu.async_copy(buf, out_hbm.at[pl.ds(base, _CHUNK_WORDS)], sem_out).wait()
        return carry

    lax.fori_loop(0, _NCHUNKS, chunk_body, 0)


def _sc_call(x_sc_flat):
    call = functools.partial(
        pl.kernel,
        out_type=jax.ShapeDtypeStruct((_B_SC * _TOTAL,), jnp.float32),
        mesh=plsc.VectorSubcoreMesh(core_axis_name="c", subcore_axis_name="s"),
        scratch_types=[
            pltpu.VMEM((_CHUNK_WORDS,), jnp.float32),
            pltpu.SemaphoreType.DMA,
            pltpu.SemaphoreType.DMA,
        ],
        compiler_params=pltpu.CompilerParams(needs_layout_passes=False),
    )(_sc_body)
    return call(x_sc_flat)


def _tc_body(x_ref, ind_ref, indt_ref, o_ref):
    x = x_ref[...]
    m = jnp.max(x, axis=1, keepdims=True)
    e = jnp.exp(x - m)
    s = jax.lax.dot_general(e, ind_ref[...], (((1,), (0,)), ((), ())),
                            preferred_element_type=jnp.float32)
    l = jnp.log(jnp.maximum(s, 1e-30))
    lb = jax.lax.dot_general(l, indt_ref[...], (((1,), (0,)), ((), ())),
                             preferred_element_type=jnp.float32)
    o_ref[...] = x - m - lb


def _tc_call(x):
    off = _B_SC // _BLK
    return pl.pallas_call(
        _tc_body,
        out_shape=jax.ShapeDtypeStruct((_BATCH, _TOTAL), jnp.float32),
        grid=(_B_TC // _BLK,),
        in_specs=[
            pl.BlockSpec((_BLK, _TOTAL), lambda i: (i + off, 0)),
            pl.BlockSpec((_TOTAL, _HPAD), lambda i: (0, 0)),
            pl.BlockSpec((_HPAD, _TOTAL), lambda i: (0, 0)),
        ],
        out_specs=pl.BlockSpec((_BLK, _TOTAL), lambda i: (i + off, 0)),
        compiler_params=pltpu.CompilerParams(
            dimension_semantics=("arbitrary",)),
    )(x, jnp.asarray(_IND), jnp.asarray(_INDT))


@jax.jit
def kernel(logits):
    sc_out = _sc_call(logits[:_B_SC].reshape(_B_SC * _TOTAL))
    tc_out = _tc_call(logits)
    return lax.dynamic_update_slice(
        tc_out, sc_out.reshape(_B_SC, _TOTAL), (0, 0))


# speedup vs baseline: 1.9873x; 1.0003x over previous
import functools

import jax
import jax.numpy as jnp
import numpy as np
from jax import lax
from jax.experimental import pallas as pl
from jax.experimental.pallas import tpu as pltpu
from jax.experimental.pallas import tpu_sc as plsc

_BATCH = 16384
_TOTAL = 2600
_SEG = 100
_NHEAD = 26
_HPAD = 32

_B_SC = 2048
_B_TC = _BATCH - _B_SC

_NWORKERS = 32
_SC_ROWS = _B_SC * _NHEAD
_ROWS_PER_W = _SC_ROWS // _NWORKERS
_CHUNK_ROWS = 208
assert _ROWS_PER_W % _CHUNK_ROWS == 0 and _CHUNK_ROWS % 16 == 0
_CHUNK_WORDS = _CHUNK_ROWS * _SEG
_NCHUNKS = _ROWS_PER_W // _CHUNK_ROWS
_GROUPS = _CHUNK_ROWS // 16

_LN2 = 0.6931471805599453
_SQRT2 = 1.4142135623730951

_BLK = 1024
_IND = np.zeros((_TOTAL, _HPAD), np.float32)
for _j in range(_TOTAL):
    _IND[_j, _j // _SEG] = 1.0
_INDT = _IND.T.copy()


def _vlog(s):
    bits = plsc.bitcast(s, jnp.int32)
    e = lax.shift_right_arithmetic(bits, 23) - 127
    mbits = jnp.bitwise_or(jnp.bitwise_and(bits, 0x007FFFFF), 0x3F800000)
    m = plsc.bitcast(mbits, jnp.float32)
    big = m > _SQRT2
    m = jnp.where(big, m * 0.5, m)
    e = (e + jnp.where(big, 1, 0)).astype(jnp.float32)
    t = (m - 1.0) / (m + 1.0)
    w = t * t
    p = 2.0 * t * (1.0 + w * (1.0 / 3.0 + w * (0.2 + w * (1.0 / 7.0 + w * (1.0 / 9.0)))))
    return e * _LN2 + p


def _compute_chunk(buf, iota16):
    def group_body(g, carry):
        base = g * (16 * _SEG)
        idx0 = base + iota16 * _SEG

        acc = [jnp.full((16,), -jnp.inf, jnp.float32) for _ in range(4)]
        for j in range(_SEG):
            v = plsc.load_gather(buf, [idx0 + j])
            acc[j % 4] = jnp.maximum(acc[j % 4], v)
        mx = jnp.maximum(jnp.maximum(acc[0], acc[1]),
                         jnp.maximum(acc[2], acc[3]))

        sacc = [jnp.zeros((16,), jnp.float32) for _ in range(4)]
        for j in range(_SEG):
            v = plsc.load_gather(buf, [idx0 + j])
            sacc[j % 4] = sacc[j % 4] + jnp.exp(v - mx)
        s = (sacc[0] + sacc[1]) + (sacc[2] + sacc[3])

        c = mx + _vlog(s)

        for j in range(_SEG):
            idx = idx0 + j
            v = plsc.load_gather(buf, [idx])
            plsc.store_scatter(buf, [idx], v - c)
        return carry

    lax.fori_loop(0, _GROUPS, group_body, 0)


def _sc_body(x_hbm, out_hbm, buf, sem_in, sem_out):
    wid = lax.axis_index("s") * 2 + lax.axis_index("c")
    wbase = wid * (_ROWS_PER_W * _SEG)
    iota16 = lax.iota(jnp.int32, 16)

    def chunk_body(t, carry):
        base = wbase + t * _CHUNK_WORDS
        pltpu.async_copy(x_hbm.at[pl.ds(base, _CHUNK_WORDS)], buf, sem_in).wait()
        _compute_chunk(buf, iota16)
        pltpu.async_copy(buf, out_hbm.at[pl.ds(base, _CHUNK_WORDS)], sem_out).wait()
        return carry

    lax.fori_loop(0, _NCHUNKS, chunk_body, 0)


def _sc_call(x_sc_flat):
    call = functools.partial(
        pl.kernel,
        out_type=jax.ShapeDtypeStruct((_B_SC * _TOTAL,), jnp.float32),
        mesh=plsc.VectorSubcoreMesh(core_axis_name="c", subcore_axis_name="s"),
        scratch_types=[
            pltpu.VMEM((_CHUNK_WORDS,), jnp.float32),
            pltpu.SemaphoreType.DMA,
            pltpu.SemaphoreType.DMA,
        ],
        compiler_params=pltpu.CompilerParams(needs_layout_passes=False),
    )(_sc_body)
    return call(x_sc_flat)


def _tc_body(x_ref, ind_ref, indt_ref, o_ref):
    x = x_ref[...]
    m = jnp.max(x, axis=1, keepdims=True)
    e = jnp.exp(x - m)
    s = jax.lax.dot_general(e, ind_ref[...], (((1,), (0,)), ((), ())),
                            preferred_element_type=jnp.float32)
    l = jnp.log(jnp.maximum(s, 1e-30))
    lb = jax.lax.dot_general(l, indt_ref[...], (((1,), (0,)), ((), ())),
                             preferred_element_type=jnp.float32)
    o_ref[...] = x - m - lb


def _tc_call(x):
    off = _B_SC // _BLK
    return pl.pallas_call(
        _tc_body,
        out_shape=jax.ShapeDtypeStruct((_BATCH, _TOTAL), jnp.float32),
        grid=(_B_TC // _BLK,),
        in_specs=[
            pl.BlockSpec((_BLK, _TOTAL), lambda i: (i + off, 0)),
            pl.BlockSpec((_TOTAL, _HPAD), lambda i: (0, 0)),
            pl.BlockSpec((_HPAD, _TOTAL), lambda i: (0, 0)),
        ],
        out_specs=pl.BlockSpec((_BLK, _TOTAL), lambda i: (i + off, 0)),
        compiler_params=pltpu.CompilerParams(
            dimension_semantics=("arbitrary",)),
    )(x, jnp.asarray(_IND), jnp.asarray(_INDT))


@jax.jit
def kernel(logits):
    sc_out = _sc_call(logits[:_B_SC].reshape(_B_SC * _TOTAL))
    tc_out = _tc_call(logits)
    return lax.dynamic_update_slice(
        tc_out, sc_out.reshape(_B_SC, _TOTAL), (0, 0))
